# Initial kernel scaffold; baseline (speedup 1.0000x reference)
import jax
import jax.numpy as jnp
from jax.experimental import pallas as pl


def kernel(x, lens, target_codes, W1, b1, w2):
    raise NotImplementedError("write your pallas kernel here")


# speedup vs baseline: 1.1055x; 1.1055x over previous
import functools

import jax
import jax.numpy as jnp
from jax import lax
from jax.experimental import pallas as pl
from jax.experimental.pallas import tpu as pltpu
from jax.experimental.pallas import tpu_sc as plsc

B, T, C = 128, 32, 10000
ATT = 64
ROWS = B * T
RB = 256
NRB = ROWS // RB


def _stream_body(x_ref, w1_ref, b1_ref, w2_ref, tc_ref,
                 y_ref, logit_ref, xg_ref):
    s = jax.nn.sigmoid(x_ref[...])
    y_ref[...] = s
    acc = jnp.dot(s, w1_ref[...], preferred_element_type=jnp.float32)
    e = jnp.tanh(acc + b1_ref[...])
    logit_ref[...] = jnp.dot(e, w2_ref[...],
                             preferred_element_type=jnp.float32)
    code_ids = lax.broadcasted_iota(jnp.int32, (RB, C), 1)
    hit = code_ids == tc_ref[...]
    xg_ref[...] = jnp.sum(jnp.where(hit, x_ref[...], 0.0), axis=1,
                          keepdims=True)


def _finish_body(logit_ref, lens_ref, xg_ref, vals_ref):
    t_ids = lax.broadcasted_iota(jnp.int32, (B, T), 1)
    mask = t_ids < lens_ref[...]
    l = jnp.where(mask, logit_ref[...], -1e9)
    m = jnp.max(l, axis=-1, keepdims=True)
    p = jnp.exp(l - m)
    score = p / jnp.sum(p, axis=-1, keepdims=True)
    vals_ref[...] = jax.nn.sigmoid(xg_ref[...] + score)


def _make_scatter():
    info = plsc.get_sparse_core_info()
    nw = info.num_cores * info.num_subcores
    per_w = ROWS // nw

    mesh = plsc.VectorSubcoreMesh(core_axis_name="c", subcore_axis_name="s")

    @functools.partial(
        pl.kernel, mesh=mesh, out_type=(),
        scratch_types=[
            pltpu.VMEM((per_w,), jnp.int32),
            pltpu.VMEM((per_w,), jnp.float32),
            pltpu.SemaphoreType.DMA,
        ],
    )
    def scatter(idx_hbm, vals_hbm, y_ref, idx_v, vals_v, sem):
        wid = lax.axis_index("s") * info.num_cores + lax.axis_index("c")
        base = wid * per_w
        pltpu.sync_copy(idx_hbm.at[pl.ds(base, per_w)], idx_v)
        pltpu.sync_copy(vals_hbm.at[pl.ds(base, per_w)], vals_v)
        pltpu.async_copy(vals_v, y_ref.at[idx_v], sem).wait()

    return scatter


_scatter = None


def kernel(x, lens, target_codes, W1, b1, w2):
    global _scatter
    if _scatter is None:
        _scatter = _make_scatter()

    x2 = x.reshape(ROWS, C)
    tc2 = target_codes.reshape(ROWS, 1)

    y2, logits, xg = pl.pallas_call(
        _stream_body,
        grid=(NRB,),
        in_specs=[
            pl.BlockSpec((RB, C), lambda r: (r, 0)),
            pl.BlockSpec((C, ATT), lambda r: (0, 0)),
            pl.BlockSpec((1, ATT), lambda r: (0, 0)),
            pl.BlockSpec((ATT, 1), lambda r: (0, 0)),
            pl.BlockSpec((RB, 1), lambda r: (r, 0)),
        ],
        out_specs=[
            pl.BlockSpec((RB, C), lambda r: (r, 0)),
            pl.BlockSpec((RB, 1), lambda r: (r, 0)),
            pl.BlockSpec((RB, 1), lambda r: (r, 0)),
        ],
        out_shape=[
            jax.ShapeDtypeStruct((ROWS, C), jnp.float32),
            jax.ShapeDtypeStruct((ROWS, 1), jnp.float32),
            jax.ShapeDtypeStruct((ROWS, 1), jnp.float32),
        ],
    )(x2, W1, b1.reshape(1, ATT), w2.reshape(ATT, 1), tc2)

    vals = pl.pallas_call(
        _finish_body,
        in_specs=[
            pl.BlockSpec((B, T), lambda: (0, 0)),
            pl.BlockSpec((B, 1), lambda: (0, 0)),
            pl.BlockSpec((B, T), lambda: (0, 0)),
        ],
        out_specs=pl.BlockSpec((B, T), lambda: (0, 0)),
        out_shape=jax.ShapeDtypeStruct((B, T), jnp.float32),
    )(logits.reshape(B, T), lens.reshape(B, 1), xg.reshape(B, T))

    flat_idx = (jnp.arange(ROWS, dtype=jnp.int32) * C
                + target_codes.reshape(ROWS))
    y_flat = y2.reshape(ROWS * C)
    y_ref = jax.new_ref(y_flat)
    _scatter(flat_idx, vals.reshape(ROWS), y_ref)
    out = jax.freeze(y_ref)
    return out.reshape(B, T, C)


# speedup vs baseline: 2.0119x; 1.8198x over previous
import jax
import jax.numpy as jnp
from jax import lax
from jax.experimental import pallas as pl

B, T, C = 128, 32, 10000
ATT = 64
ROWS = B * T
RB = 256
NRB = ROWS // RB


def _fused_body(x_ref, w1_ref, b1_ref, w2_ref, tc_ref, msk_ref, out_ref):
    x = x_ref[...]
    s = jax.nn.sigmoid(x)
    acc = jnp.dot(s, w1_ref[...], preferred_element_type=jnp.float32)
    e = jnp.tanh(acc + b1_ref[...])
    logits = jnp.dot(e, w2_ref[...],
                     preferred_element_type=jnp.float32)
    l = jnp.where(msk_ref[...] > 0, logits, -30.0)
    p = jnp.exp(l)
    ri = lax.broadcasted_iota(jnp.int32, (RB, RB), 0) // T
    ci = lax.broadcasted_iota(jnp.int32, (RB, RB), 1) // T
    g = (ri == ci).astype(jnp.float32)
    gsum = jnp.dot(g, p, preferred_element_type=jnp.float32)
    score = p / gsum
    code_ids = lax.broadcasted_iota(jnp.int32, (RB, C), 1)
    hit = code_ids == tc_ref[...]
    xg = jnp.sum(jnp.where(hit, x, 0.0), axis=1, keepdims=True)
    vals = jax.nn.sigmoid(xg + score)
    out_ref[...] = jnp.where(hit, vals, s)


def kernel(x, lens, target_codes, W1, b1, w2):
    x2 = x.reshape(ROWS, C)
    tc2 = target_codes.reshape(ROWS, 1)
    t_of_row = jnp.tile(jnp.arange(T, dtype=jnp.int32), B)
    msk = (t_of_row < jnp.repeat(lens, T)).astype(jnp.int32).reshape(ROWS, 1)
    out2 = pl.pallas_call(
        _fused_body,
        grid=(NRB,),
        in_specs=[
            pl.BlockSpec((RB, C), lambda r: (r, 0)),
            pl.BlockSpec((C, ATT), lambda r: (0, 0)),
            pl.BlockSpec((1, ATT), lambda r: (0, 0)),
            pl.BlockSpec((ATT, 1), lambda r: (0, 0)),
            pl.BlockSpec((RB, 1), lambda r: (r, 0)),
            pl.BlockSpec((RB, 1), lambda r: (r, 0)),
        ],
        out_specs=pl.BlockSpec((RB, C), lambda r: (r, 0)),
        out_shape=jax.ShapeDtypeStruct((ROWS, C), jnp.float32),
    )(x2, W1, b1.reshape(1, ATT), w2.reshape(ATT, 1), tc2, msk)
    return out2.reshape(B, T, C)
